# baseline (speedup 1.0000x reference)
import functools

import jax
import jax.numpy as jnp
from jax import lax
from jax.experimental import pallas as pl
from jax.experimental.pallas import tpu as pltpu
from jax.experimental.pallas import tpu_sc as plsc

B, T, D = 4, 8192, 768
P, LMAX, C = 4096, 8, 64
CAT = 2 * C
NC, NS = 2, 16
NW = NC * NS
CHUNK = 16
RPC = CHUNK * LMAX

_NEG = -1e30


def _mm_body(x_ref, w_ref, o_ref):
    o_ref[...] = jnp.dot(x_ref[...], w_ref[...],
                         preferred_element_type=jnp.float32)


def _project(x, w_cat):
    bm = 2048
    return pl.pallas_call(
        _mm_body,
        grid=(x.shape[0] // bm,),
        in_specs=[pl.BlockSpec((bm, D), lambda i: (i, 0)),
                  pl.BlockSpec((D, CAT), lambda i: (0, 0))],
        out_specs=pl.BlockSpec((bm, CAT), lambda i: (i, 0)),
        out_shape=jax.ShapeDtypeStruct((x.shape[0], CAT), jnp.float32),
    )(x, w_cat)


_mesh = plsc.VectorSubcoreMesh(core_axis_name="c", subcore_axis_name="s")


def _make_sc_pool(nb):
    bp = nb * P
    ppw = bp // NW
    nchunk = ppw // CHUNK
    wpb = NW // nb

    @functools.partial(
        pl.kernel,
        mesh=_mesh,
        out_type=jax.ShapeDtypeStruct((nb, P, C), jnp.float32),
        scratch_types=[
            pltpu.VMEM((ppw * LMAX,), jnp.int32),
            pltpu.VMEM((ppw, LMAX * 16), jnp.float32),
            pltpu.VMEM((LMAX * C,), jnp.float32),
            pltpu.VMEM((2, RPC, CAT), jnp.float32),
            pltpu.VMEM((2, CHUNK, C), jnp.float32),
            pltpu.SemaphoreType.DMA,
            pltpu.SemaphoreType.DMA,
            pltpu.SemaphoreType.DMA,
            pltpu.SemaphoreType.DMA,
        ],
    )
    def _sc_pool(cat_hbm, idx_hbm, mb_hbm, bpos_hbm, out_hbm,
                 idx_v, mb_v, bpos_v, rows_v, out_v,
                 gsem0, gsem1, osem0, osem1):
        wid = lax.axis_index("s") * NC + lax.axis_index("c")
        bb = wid // wpb
        tok_off = bb * T
        start_w = wid * ppw
        pstart_w = start_w - bb * P
        gsem = (gsem0, gsem1)
        osem = (osem0, osem1)

        pltpu.sync_copy(bpos_hbm, bpos_v)
        pltpu.sync_copy(idx_hbm.at[pl.ds(start_w * LMAX, ppw * LMAX)], idx_v)
        pltpu.sync_copy(mb_hbm.at[pl.ds(start_w, ppw)], mb_v)

        def add_off(i, carry):
            sl = pl.ds(16 * i, 16)
            idx_v[sl] = idx_v[sl] + tok_off
            return carry

        lax.fori_loop(0, ppw * LMAX // 16, add_off, 0)

        bpos = [[bpos_v[pl.ds(l * C + 16 * j, 16)] for j in range(C // 16)]
                for l in range(LMAX)]

        def _gather(ci, buf):
            idx_slice = idx_v.at[pl.ds(ci * RPC, RPC)]
            return pltpu.async_copy(cat_hbm.at[idx_slice], rows_v.at[buf],
                                    gsem[buf])

        _gather(0, 0)

        def pair_body(g, carry):
            for bu in range(2):
                ci = 2 * g + bu
                nci = jnp.minimum(ci + 1, nchunk - 1)
                _gather(nci, 1 - bu)
                pltpu.make_async_copy(
                    cat_hbm.at[idx_v.at[pl.ds(ci * RPC, RPC)]],
                    rows_v.at[bu], gsem[bu]).wait()

                @pl.when(ci >= 2)
                def _():
                    pltpu.make_async_copy(
                        out_v.at[bu],
                        out_hbm.at[bb, pl.ds(pstart_w + (ci - 2) * CHUNK,
                                             CHUNK)],
                        osem[bu]).wait()

                @plsc.parallel_loop(0, CHUNK, unroll=4)
                def phrase_body(p):
                    base = p * LMAX
                    mrow = ci * CHUNK + p
                    mb = [mb_v[mrow, pl.ds(16 * l, 16)] for l in range(LMAX)]
                    for j in range(C // 16):
                        e = [jnp.exp(rows_v[bu, base + l,
                                            pl.ds(C + 16 * j, 16)]
                                     + bpos[l][j] + mb[l])
                             for l in range(LMAX)]
                        s = e[0]
                        for l in range(1, LMAX):
                            s = s + e[l]
                        acc = e[0] * rows_v[bu, base, pl.ds(16 * j, 16)]
                        for l in range(1, LMAX):
                            acc = acc + e[l] * rows_v[bu, base + l,
                                                      pl.ds(16 * j, 16)]
                        out_v[bu, p, pl.ds(16 * j, 16)] = acc / s

                pltpu.async_copy(
                    out_v.at[bu],
                    out_hbm.at[bb, pl.ds(pstart_w + ci * CHUNK, CHUNK)],
                    osem[bu])
            return carry

        lax.fori_loop(0, nchunk // 2, pair_body, 0)

        pltpu.make_async_copy(
            cat_hbm.at[idx_v.at[pl.ds((nchunk - 1) * RPC, RPC)]],
            rows_v.at[0], gsem[0]).wait()
        for bu in range(2):
            ci = nchunk - 2 + bu
            pltpu.make_async_copy(
                out_v.at[bu],
                out_hbm.at[bb, pl.ds(pstart_w + ci * CHUNK, CHUNK)],
                osem[bu]).wait()

    return _sc_pool


_sc_pool_half = _make_sc_pool(B // 2)


def kernel(h, phrase_mask, phrase_token_idx, W_kv, W_z, B_pos):
    w_cat = jnp.concatenate([W_kv, W_z], axis=1)
    bpos_flat = B_pos.astype(jnp.float32).reshape(-1)
    idx = phrase_token_idx.astype(jnp.int32).reshape(B, P * LMAX)
    mb = jnp.where(phrase_mask, 0.0, _NEG).astype(jnp.float32)
    mb_exp = jnp.broadcast_to(
        mb[:, :, :, None], (B, P, LMAX, 16)).reshape(B, P, LMAX * 16)

    hb = B // 2
    outs = []
    for half in range(2):
        x = h[half * hb:(half + 1) * hb].reshape(hb * T, D)
        cat = _project(x, w_cat)
        outs.append(_sc_pool_half(
            cat,
            idx[half * hb:(half + 1) * hb].reshape(-1),
            mb_exp[half * hb:(half + 1) * hb].reshape(hb * P, LMAX * 16),
            bpos_flat))
    return jnp.concatenate(outs, axis=0)


# speedup vs baseline: 1.0088x; 1.0088x over previous
import functools

import jax
import jax.numpy as jnp
from jax import lax
from jax.experimental import pallas as pl
from jax.experimental.pallas import tpu as pltpu
from jax.experimental.pallas import tpu_sc as plsc

B, T, D = 4, 8192, 768
P, LMAX, C = 4096, 8, 64
CAT = 2 * C
NC, NS = 2, 16
NW = NC * NS
CHUNK = 16
RPC = CHUNK * LMAX

_NEG = -1e30


def _mm_body(x_ref, w_ref, o_ref):
    o_ref[...] = jnp.dot(x_ref[...], w_ref[...],
                         preferred_element_type=jnp.float32)


def _project(x, w_cat):
    bm = 2048
    return pl.pallas_call(
        _mm_body,
        grid=(x.shape[0] // bm,),
        in_specs=[pl.BlockSpec((bm, D), lambda i: (i, 0)),
                  pl.BlockSpec((D, CAT), lambda i: (0, 0))],
        out_specs=pl.BlockSpec((bm, CAT), lambda i: (i, 0)),
        out_shape=jax.ShapeDtypeStruct((x.shape[0], CAT), jnp.float32),
    )(x, w_cat)


_mesh = plsc.VectorSubcoreMesh(core_axis_name="c", subcore_axis_name="s")


def _make_sc_pool(nb):
    bp = nb * P
    ppw = bp // NW
    nchunk = ppw // CHUNK
    wpb = NW // nb

    @functools.partial(
        pl.kernel,
        mesh=_mesh,
        out_type=jax.ShapeDtypeStruct((nb, P, C), jnp.float32),
        scratch_types=[
            pltpu.VMEM((ppw * LMAX,), jnp.int32),
            pltpu.VMEM((ppw, LMAX * 16), jnp.float32),
            pltpu.VMEM((LMAX * C,), jnp.float32),
            pltpu.VMEM((2, RPC, CAT), jnp.float32),
            pltpu.VMEM((2, CHUNK, C), jnp.float32),
            pltpu.SemaphoreType.DMA,
            pltpu.SemaphoreType.DMA,
            pltpu.SemaphoreType.DMA,
            pltpu.SemaphoreType.DMA,
        ],
    )
    def _sc_pool(cat_hbm, idx_hbm, mb_hbm, bpos_hbm, out_hbm,
                 idx_v, mb_v, bpos_v, rows_v, out_v,
                 gsem0, gsem1, osem0, osem1):
        wid = lax.axis_index("s") * NC + lax.axis_index("c")
        bb = wid // wpb
        tok_off = bb * T
        start_w = wid * ppw
        pstart_w = start_w - bb * P
        gsem = (gsem0, gsem1)
        osem = (osem0, osem1)

        pltpu.sync_copy(bpos_hbm, bpos_v)
        pltpu.sync_copy(idx_hbm.at[pl.ds(start_w * LMAX, ppw * LMAX)], idx_v)
        pltpu.sync_copy(mb_hbm.at[pl.ds(start_w, ppw)], mb_v)

        def add_off(i, carry):
            sl = pl.ds(16 * i, 16)
            idx_v[sl] = idx_v[sl] + tok_off
            return carry

        lax.fori_loop(0, ppw * LMAX // 16, add_off, 0)

        bpos = [[bpos_v[pl.ds(l * C + 16 * j, 16)] for j in range(C // 16)]
                for l in range(LMAX)]

        def _gather(ci, buf):
            idx_slice = idx_v.at[pl.ds(ci * RPC, RPC)]
            return pltpu.async_copy(cat_hbm.at[idx_slice], rows_v.at[buf],
                                    gsem[buf])

        _gather(0, 0)

        def pair_body(g, carry):
            for bu in range(2):
                ci = 2 * g + bu
                nci = jnp.minimum(ci + 1, nchunk - 1)
                _gather(nci, 1 - bu)
                pltpu.make_async_copy(
                    cat_hbm.at[idx_v.at[pl.ds(ci * RPC, RPC)]],
                    rows_v.at[bu], gsem[bu]).wait()

                @pl.when(ci >= 2)
                def _():
                    pltpu.make_async_copy(
                        out_v.at[bu],
                        out_hbm.at[bb, pl.ds(pstart_w + (ci - 2) * CHUNK,
                                             CHUNK)],
                        osem[bu]).wait()

                @plsc.parallel_loop(0, CHUNK, unroll=4)
                def phrase_body(p):
                    base = p * LMAX
                    mrow = ci * CHUNK + p
                    mb = [mb_v[mrow, pl.ds(16 * l, 16)] for l in range(LMAX)]
                    for j in range(C // 16):
                        e = [jnp.exp(rows_v[bu, base + l,
                                            pl.ds(C + 16 * j, 16)]
                                     + bpos[l][j] + mb[l])
                             for l in range(LMAX)]
                        s = e[0]
                        for l in range(1, LMAX):
                            s = s + e[l]
                        acc = e[0] * rows_v[bu, base, pl.ds(16 * j, 16)]
                        for l in range(1, LMAX):
                            acc = acc + e[l] * rows_v[bu, base + l,
                                                      pl.ds(16 * j, 16)]
                        out_v[bu, p, pl.ds(16 * j, 16)] = acc / s

                pltpu.async_copy(
                    out_v.at[bu],
                    out_hbm.at[bb, pl.ds(pstart_w + ci * CHUNK, CHUNK)],
                    osem[bu])
            return carry

        lax.fori_loop(0, nchunk // 2, pair_body, 0)

        pltpu.make_async_copy(
            cat_hbm.at[idx_v.at[pl.ds((nchunk - 1) * RPC, RPC)]],
            rows_v.at[0], gsem[0]).wait()
        for bu in range(2):
            ci = nchunk - 2 + bu
            pltpu.make_async_copy(
                out_v.at[bu],
                out_hbm.at[bb, pl.ds(pstart_w + ci * CHUNK, CHUNK)],
                osem[bu]).wait()

    return _sc_pool


_sc_pool_half = _make_sc_pool(B // 2)


def kernel(h, phrase_mask, phrase_token_idx, W_kv, W_z, B_pos):
    w_cat = jnp.concatenate([W_kv, W_z], axis=1)
    bpos_flat = B_pos.astype(jnp.float32).reshape(-1)

    hb = B // 2
    outs = []
    for half in range(2):
        sl = slice(half * hb, (half + 1) * hb)
        x = h[sl].reshape(hb * T, D)
        cat = _project(x, w_cat)
        idx_h = phrase_token_idx[sl].astype(jnp.int32).reshape(-1)
        mb_h = jnp.where(phrase_mask[sl], 0.0, _NEG).astype(jnp.float32)
        mb_exp_h = jnp.broadcast_to(
            mb_h[:, :, :, None],
            (hb, P, LMAX, 16)).reshape(hb * P, LMAX * 16)
        outs.append(_sc_pool_half(cat, idx_h, mb_exp_h, bpos_flat))
    return jnp.concatenate(outs, axis=0)


# speedup vs baseline: 1.3877x; 1.3757x over previous
import functools

import jax
import jax.numpy as jnp
from jax import lax
from jax.experimental import pallas as pl
from jax.experimental.pallas import tpu as pltpu
from jax.experimental.pallas import tpu_sc as plsc

B, T, D = 4, 8192, 768
P, LMAX, C = 4096, 8, 64
CAT = 2 * C
NC, NS = 2, 16
NW = NC * NS
CHUNK = 16
RPC = CHUNK * LMAX

_NEG = -1e30


def _mm_body(x_ref, w_ref, o_ref):
    o_ref[...] = jnp.dot(x_ref[...], w_ref[...],
                         preferred_element_type=jnp.float32)


def _project_rows(x, w_cat, row0, nrows):
    bm = 2048
    blk0 = row0 // bm
    return pl.pallas_call(
        _mm_body,
        grid=(nrows // bm,),
        in_specs=[pl.BlockSpec((bm, D), lambda i: (blk0 + i, 0)),
                  pl.BlockSpec((D, CAT), lambda i: (0, 0))],
        out_specs=pl.BlockSpec((bm, CAT), lambda i: (i, 0)),
        out_shape=jax.ShapeDtypeStruct((nrows, CAT), jnp.float32),
    )(x, w_cat)


_mesh = plsc.VectorSubcoreMesh(core_axis_name="c", subcore_axis_name="s")


def _make_sc_pool(nb):
    bp = nb * P
    ppw = bp // NW
    nchunk = ppw // CHUNK
    wpb = NW // nb

    @functools.partial(
        pl.kernel,
        mesh=_mesh,
        out_type=jax.ShapeDtypeStruct((nb, P, C), jnp.float32),
        scratch_types=[
            pltpu.VMEM((ppw * LMAX,), jnp.int32),
            pltpu.VMEM((ppw, LMAX * 16), jnp.float32),
            pltpu.VMEM((LMAX * C,), jnp.float32),
            pltpu.VMEM((2, RPC, CAT), jnp.float32),
            pltpu.VMEM((2, CHUNK, C), jnp.float32),
            pltpu.SemaphoreType.DMA,
            pltpu.SemaphoreType.DMA,
            pltpu.SemaphoreType.DMA,
            pltpu.SemaphoreType.DMA,
        ],
    )
    def _sc_pool(cat_hbm, idx_hbm, mb_hbm, bpos_hbm, out_hbm,
                 idx_v, mb_v, bpos_v, rows_v, out_v,
                 gsem0, gsem1, osem0, osem1):
        wid = lax.axis_index("s") * NC + lax.axis_index("c")
        bb = wid // wpb
        tok_off = bb * T
        start_w = wid * ppw
        pstart_w = start_w - bb * P
        gsem = (gsem0, gsem1)
        osem = (osem0, osem1)

        pltpu.sync_copy(bpos_hbm, bpos_v)
        pltpu.sync_copy(idx_hbm.at[pl.ds(start_w * LMAX, ppw * LMAX)], idx_v)
        pltpu.sync_copy(mb_hbm.at[pl.ds(start_w, ppw)], mb_v)

        def add_off(i, carry):
            sl = pl.ds(16 * i, 16)
            idx_v[sl] = idx_v[sl] + tok_off
            return carry

        lax.fori_loop(0, ppw * LMAX // 16, add_off, 0)

        bpos = [[bpos_v[pl.ds(l * C + 16 * j, 16)] for j in range(C // 16)]
                for l in range(LMAX)]

        def _gather(ci, buf):
            idx_slice = idx_v.at[pl.ds(ci * RPC, RPC)]
            return pltpu.async_copy(cat_hbm.at[idx_slice], rows_v.at[buf],
                                    gsem[buf])

        _gather(0, 0)

        def pair_body(g, carry):
            for bu in range(2):
                ci = 2 * g + bu
                nci = jnp.minimum(ci + 1, nchunk - 1)
                _gather(nci, 1 - bu)
                pltpu.make_async_copy(
                    cat_hbm.at[idx_v.at[pl.ds(ci * RPC, RPC)]],
                    rows_v.at[bu], gsem[bu]).wait()

                @pl.when(ci >= 2)
                def _():
                    pltpu.make_async_copy(
                        out_v.at[bu],
                        out_hbm.at[bb, pl.ds(pstart_w + (ci - 2) * CHUNK,
                                             CHUNK)],
                        osem[bu]).wait()

                @plsc.parallel_loop(0, CHUNK, unroll=4)
                def phrase_body(p):
                    base = p * LMAX
                    mrow = ci * CHUNK + p
                    mb = [mb_v[mrow, pl.ds(16 * l, 16)] for l in range(LMAX)]
                    for j in range(C // 16):
                        e = [jnp.exp(rows_v[bu, base + l,
                                            pl.ds(C + 16 * j, 16)]
                                     + bpos[l][j] + mb[l])
                             for l in range(LMAX)]
                        s = e[0]
                        for l in range(1, LMAX):
                            s = s + e[l]
                        acc = e[0] * rows_v[bu, base, pl.ds(16 * j, 16)]
                        for l in range(1, LMAX):
                            acc = acc + e[l] * rows_v[bu, base + l,
                                                      pl.ds(16 * j, 16)]
                        out_v[bu, p, pl.ds(16 * j, 16)] = acc / s

                pltpu.async_copy(
                    out_v.at[bu],
                    out_hbm.at[bb, pl.ds(pstart_w + ci * CHUNK, CHUNK)],
                    osem[bu])
            return carry

        lax.fori_loop(0, nchunk // 2, pair_body, 0)

        pltpu.make_async_copy(
            cat_hbm.at[idx_v.at[pl.ds((nchunk - 1) * RPC, RPC)]],
            rows_v.at[0], gsem[0]).wait()
        for bu in range(2):
            ci = nchunk - 2 + bu
            pltpu.make_async_copy(
                out_v.at[bu],
                out_hbm.at[bb, pl.ds(pstart_w + ci * CHUNK, CHUNK)],
                osem[bu]).wait()

    return _sc_pool


_sc_pool_half = _make_sc_pool(B // 2)


def kernel(h, phrase_mask, phrase_token_idx, W_kv, W_z, B_pos):
    w_cat = jnp.concatenate([W_kv, W_z], axis=1)
    bpos_flat = B_pos.astype(jnp.float32).reshape(-1)

    hb = B // 2
    x = h.reshape(B * T, D)
    outs = []
    for half in range(2):
        sl = slice(half * hb, (half + 1) * hb)
        cat = _project_rows(x, w_cat, half * hb * T, hb * T)
        idx_h = phrase_token_idx[sl].astype(jnp.int32).reshape(-1)
        mb_h = jnp.where(phrase_mask[sl], 0.0, _NEG).astype(jnp.float32)
        mb_exp_h = jnp.broadcast_to(
            mb_h[:, :, :, None],
            (hb, P, LMAX, 16)).reshape(hb * P, LMAX * 16)
        outs.append(_sc_pool_half(cat, idx_h, mb_exp_h, bpos_flat))
    return jnp.concatenate(outs, axis=0)
